# baseline (speedup 1.0000x reference)
import functools

import jax
import jax.numpy as jnp
from jax import lax
from jax.experimental import pallas as pl
from jax.experimental.pallas import tpu as pltpu
from jax.experimental.pallas import tpu_sc as plsc

B = 4
D = 768
N = 1000

_MESH = plsc.VectorSubcoreMesh(core_axis_name="c", subcore_axis_name="s")


@functools.partial(
    pl.kernel,
    out_type=jax.ShapeDtypeStruct((B, D), jnp.float32),
    mesh=_MESH,
    scratch_types=[
        pltpu.VMEM((B,), jnp.int32),
        pltpu.VMEM((B, D), jnp.float32),
        pltpu.SemaphoreType.DMA,
    ],
)
def _sc_gather(idx_hbm, table_hbm, out_hbm, idx_v, rows_v, sem):
    wid = lax.axis_index("s") * 2 + lax.axis_index("c")

    @pl.when(wid == 0)
    def _():
        pltpu.sync_copy(idx_hbm, idx_v)
        pltpu.async_copy(table_hbm.at[idx_v], rows_v, sem).wait()
        pltpu.sync_copy(rows_v, out_hbm)


def _dense_body(x_ref, ws_ref, bs_ref, wc_ref, bc_ref, out_ref):
    x = x_ref[...]
    h = jnp.tanh(
        jax.lax.dot_general(x, ws_ref[...], (((1,), (0,)), ((), ())),
                            preferred_element_type=jnp.float32)
        + bs_ref[...]
    )
    y = x + h
    out_ref[...] = (
        jax.lax.dot_general(y, wc_ref[...], (((1,), (0,)), ((), ())),
                            preferred_element_type=jnp.float32)
        + bc_ref[...]
    )


def kernel(tokens, embed_table, W_s, b_s, W_c, b_c):
    last = tokens[:, -1].astype(jnp.int32)
    x = _sc_gather(last, embed_table)
    logits = pl.pallas_call(
        _dense_body,
        in_specs=[
            pl.BlockSpec((B, D), lambda: (0, 0)),
            pl.BlockSpec((D, D), lambda: (0, 0)),
            pl.BlockSpec((1, D), lambda: (0, 0)),
            pl.BlockSpec((D, N), lambda: (0, 0)),
            pl.BlockSpec((1, N), lambda: (0, 0)),
        ],
        out_specs=pl.BlockSpec((B, N), lambda: (0, 0)),
        out_shape=jax.ShapeDtypeStruct((B, N), jnp.float32),
    )(x, W_s, b_s.reshape(1, D), W_c, b_c.reshape(1, N))
    return (logits, None)


# speedup vs baseline: 2.2669x; 2.2669x over previous
import jax
import jax.numpy as jnp
from jax.experimental import pallas as pl
from jax.experimental.pallas import tpu as pltpu

B = 4
D = 768
N = 1000


def _body(idx_ref, emb_hbm, ws_ref, bs_ref, wc_ref, bc_ref, out_ref,
          x_ref, sems):
    for i in range(B):
        pltpu.make_async_copy(
            emb_hbm.at[pl.ds(idx_ref[i], 1), :],
            x_ref.at[pl.ds(i, 1), :],
            sems.at[i],
        ).start()
    for i in range(B):
        pltpu.make_async_copy(
            emb_hbm.at[pl.ds(idx_ref[i], 1), :],
            x_ref.at[pl.ds(i, 1), :],
            sems.at[i],
        ).wait()
    x = x_ref[...]
    h = jnp.tanh(
        jax.lax.dot_general(x, ws_ref[...], (((1,), (0,)), ((), ())),
                            preferred_element_type=jnp.float32)
        + bs_ref[...]
    )
    y = x + h
    out_ref[...] = (
        jax.lax.dot_general(y, wc_ref[...], (((1,), (0,)), ((), ())),
                            preferred_element_type=jnp.float32)
        + bc_ref[...]
    )


def kernel(tokens, embed_table, W_s, b_s, W_c, b_c):
    last = tokens[:, -1].astype(jnp.int32)
    bs2 = b_s.reshape(1, D)
    bc2 = b_c.reshape(1, N)
    logits = pl.pallas_call(
        _body,
        in_specs=[
            pl.BlockSpec(memory_space=pltpu.SMEM),
            pl.BlockSpec(memory_space=pl.ANY),
            pl.BlockSpec((D, D), lambda: (0, 0)),
            pl.BlockSpec((1, D), lambda: (0, 0)),
            pl.BlockSpec((D, N), lambda: (0, 0)),
            pl.BlockSpec((1, N), lambda: (0, 0)),
        ],
        out_specs=pl.BlockSpec((B, N), lambda: (0, 0)),
        out_shape=jax.ShapeDtypeStruct((B, N), jnp.float32),
        scratch_shapes=[
            pltpu.VMEM((B, D), jnp.float32),
            pltpu.SemaphoreType.DMA((B,)),
        ],
    )(last, embed_table, W_s, bs2, W_c, bc2)
    return (logits, None)
